# Initial kernel scaffold; baseline (speedup 1.0000x reference)
import jax
import jax.numpy as jnp
from jax.experimental import pallas as pl


def kernel(vision_feat, vision_pos, motion_feat, motion_pos, text_word_features, text_pos, lin_w1, att1, rel_w1, root_w1, bias1, lin_w2, att2, rel_w2, root_w2, bias2, bn_gamma, bn_beta):
    raise NotImplementedError("write your pallas kernel here")


# speedup vs baseline: 3.9636x; 3.9636x over previous
import numpy as np
import jax
import jax.numpy as jnp
from jax.experimental import pallas as pl
from jax.experimental.pallas import tpu as pltpu

_H = 4
_NUM_REL = 7
_FRAMES = 4
_BS = 2
_N_NODES = _BS * (2 * _FRAMES + 2)


def _build_static_edges():
    ei = []
    et = []
    for b in range(_BS):
        start = b * (2 * _FRAMES + 1)
        total = start + 2 * _FRAMES + 1
        first_flow = start + _FRAMES
        for img in range(start, first_flow):
            j = img - start
            if img < first_flow - 1:
                ei.append([img, img + 1]); ei.append([first_flow + j, first_flow + j + 1])
                et += [0, 1]
            ei.append([img, first_flow + j]); ei.append([first_flow + j, img]); et += [2, 2]
            ei.append([img, total - 1]); ei.append([total - 1, img]); et += [3, 3]
            ei.append([first_flow + j, total - 1]); ei.append([total - 1, first_flow + j]); et += [4, 4]
        g = total
        for node in range(start, total):
            ei.append([g, node]); ei.append([node, g]); et += [5, 5]
    ei = np.array(ei, dtype=np.int32).T
    n_used = int(ei.max()) + 1
    loops = np.tile(np.arange(n_used, dtype=np.int32), (2, 1))
    ei = np.concatenate([ei, loops], axis=1)
    et = np.concatenate([np.array(et, dtype=np.int32),
                         np.full((n_used,), 6, dtype=np.int32)])
    src, dst = ei[0], ei[1]

    cnt = np.zeros((_N_NODES, _NUM_REL), np.int64)
    for s, d, t in zip(src, dst, et):
        cnt[d, t] += 1
    norm = 1.0 / np.maximum(cnt[dst, et], 1)

    present = set(dst.tolist())
    extra = [d for d in range(_N_NODES) if d not in present]
    src = np.concatenate([src, np.array(extra, np.int32)])
    dst = np.concatenate([dst, np.array(extra, np.int32)])
    et = np.concatenate([et, np.zeros(len(extra), np.int32)])
    norm = np.concatenate([norm, np.zeros(len(extra))])

    order = np.argsort(dst, kind="stable")
    src, dst, et, norm = src[order], dst[order], et[order], norm[order]

    fv = np.ones(len(dst), np.int32)
    fv[1:] = (dst[1:] != dst[:-1]).astype(np.int32)
    seg = []
    starts = np.flatnonzero(fv)
    for i, s0 in enumerate(starts):
        s1 = starts[i + 1] if i + 1 < len(starts) else len(dst)
        seg.append((int(s0), int(s1 - s0)))
    return (src.astype(np.int32), dst.astype(np.int32), et.astype(np.int32),
            norm.astype(np.float32), fv, seg)


_SRC, _DST, _ET, _NORM, _FV, _SEG = _build_static_edges()
_E = len(_SRC)


def _prep_body(xraw_ref, out_ref):
    d = xraw_ref[...] + xraw_ref[...]
    out_ref[:, : 2 * _FRAMES + 1] = d
    g = d[:, 0:1]
    for i in range(1, 2 * _FRAMES + 1):
        g = g + d[:, i:i + 1]
    out_ref[:, 2 * _FRAMES + 1: 2 * _FRAMES + 2] = g * (1.0 / (2 * _FRAMES + 1))


def _prep(xraw):
    bs, m, c, f = xraw.shape
    return pl.pallas_call(
        _prep_body,
        out_shape=jax.ShapeDtypeStruct((bs, m + 1, c, f), jnp.float32),
    )(xraw)


def _proj_body(x_ref, lin_ref, root_ref, bias_ref, xl_ref, init_ref, *, o, relu_in):
    x = x_ref[0]
    if relu_in:
        x = jnp.maximum(x, 0.0)
    lin = lin_ref[...]
    xl = jnp.dot(lin, x, preferred_element_type=jnp.float32)
    xl = xl.reshape(_H, o, xl.shape[-1])
    xl_ref[0] = xl
    xr = (xl[0] + xl[1] + xl[2] + xl[3]) * (1.0 / _H)
    root = root_ref[...]
    init = jnp.dot(root.T, xr, preferred_element_type=jnp.float32)
    init_ref[0] = init + bias_ref[...]


def _project(x, lin_w, root_w, bias, relu_in):
    n, c, f = x.shape
    k = lin_w.shape[0]
    o = k // _H
    import functools
    body = functools.partial(_proj_body, o=o, relu_in=relu_in)
    return pl.pallas_call(
        body,
        grid=(n,),
        in_specs=[
            pl.BlockSpec((1, c, f), lambda i: (i, 0, 0)),
            pl.BlockSpec((k, c), lambda i: (0, 0)),
            pl.BlockSpec((o, o), lambda i: (0, 0)),
            pl.BlockSpec((o, 1), lambda i: (0, 0)),
        ],
        out_specs=[
            pl.BlockSpec((1, _H, o, f), lambda i: (i, 0, 0, 0)),
            pl.BlockSpec((1, o, f), lambda i: (i, 0, 0)),
        ],
        out_shape=[
            jax.ShapeDtypeStruct((n, _H, o, f), jnp.float32),
            jax.ShapeDtypeStruct((n, o, f), jnp.float32),
        ],
    )(x, lin_w, root_w, bias.reshape(o, 1))


def _logits_body(src_ref, dst_ref, xl_ref, att_ref, out_ref):
    e = pl.program_id(0)
    s = src_ref[e]
    d = dst_ref[e]
    xs = xl_ref[pl.ds(s, 1)][0] + xl_ref[pl.ds(d, 1)][0]
    lr = jnp.where(xs >= 0, xs, 0.2 * xs)
    att = att_ref[...]
    out_ref[0] = jnp.sum(lr * att[:, :, None], axis=1)


def _edge_logits(xl, att, src, dst):
    n, h, o, f = xl.shape
    grid_spec = pltpu.PrefetchScalarGridSpec(
        num_scalar_prefetch=2,
        grid=(_E,),
        in_specs=[
            pl.BlockSpec((n, h, o, f), lambda e, sr, dr: (0, 0, 0, 0)),
            pl.BlockSpec((h, o), lambda e, sr, dr: (0, 0)),
        ],
        out_specs=pl.BlockSpec((1, h, f), lambda e, sr, dr: (e, 0, 0)),
    )
    return pl.pallas_call(
        _logits_body,
        grid_spec=grid_spec,
        out_shape=jax.ShapeDtypeStruct((_E, h, f), jnp.float32),
    )(src, dst, xl, att)


def _softmax_body(logit_ref, aw_ref):
    for s0, cnt in _SEG:
        rows = [logit_ref[s0 + i] for i in range(cnt)]
        amax = rows[0]
        for r in rows[1:]:
            amax = jnp.maximum(amax, r)
        exs = [jnp.exp(r - amax) for r in rows]
        den = exs[0]
        for r in exs[1:]:
            den = den + r
        inv = 1.0 / (den + 1e-16)
        for i in range(cnt):
            nv = float(_NORM[s0 + i]) / _H
            aw_ref[s0 + i] = exs[i] * inv * nv


def _segment_softmax(logit):
    return pl.pallas_call(
        _softmax_body,
        out_shape=jax.ShapeDtypeStruct(logit.shape, jnp.float32),
    )(logit)


def _message_body(src_ref, et_ref, fv_ref, dst_ref, xl_ref, aw_ref, relw_ref,
                  init_ref, out_ref):
    e = pl.program_id(0)
    s = src_ref[e]
    t = et_ref[e]
    xj = xl_ref[pl.ds(s, 1)][0]
    a = aw_ref[0]
    m = xj[0] * a[0][None, :]
    for hh in range(1, _H):
        m = m + xj[hh] * a[hh][None, :]
    w = relw_ref[pl.ds(t, 1)][0]
    msg = jax.lax.dot_general(w, m, (((0,), (0,)), ((), ())),
                              preferred_element_type=jnp.float32)

    @pl.when(fv_ref[e] == 1)
    def _():
        out_ref[0] = init_ref[0] + msg

    @pl.when(fv_ref[e] == 0)
    def _():
        out_ref[0] = out_ref[0] + msg


def _messages(xl, aw, rel_w, init, src, et, fv, dst):
    n, h, o, f = xl.shape
    grid_spec = pltpu.PrefetchScalarGridSpec(
        num_scalar_prefetch=4,
        grid=(_E,),
        in_specs=[
            pl.BlockSpec((n, h, o, f), lambda e, s1, s2, s3, s4: (0, 0, 0, 0)),
            pl.BlockSpec((1, h, f), lambda e, s1, s2, s3, s4: (e, 0, 0)),
            pl.BlockSpec((_NUM_REL, o, o), lambda e, s1, s2, s3, s4: (0, 0, 0)),
            pl.BlockSpec((1, o, f), lambda e, s1, s2, s3, s4: (s4[e], 0, 0)),
        ],
        out_specs=pl.BlockSpec((1, o, f), lambda e, s1, s2, s3, s4: (s4[e], 0, 0)),
    )
    return pl.pallas_call(
        _message_body,
        grid_spec=grid_spec,
        out_shape=jax.ShapeDtypeStruct((n, o, f), jnp.float32),
    )(src, et, fv, dst, xl, aw, rel_w, init)


def _bn_body(y_ref, g_ref, b_ref, out_ref):
    y = y_ref[...]
    n, c, f = y.shape
    cnt = n * f
    mu = jnp.sum(jnp.sum(y, axis=2), axis=0) * (1.0 / cnt)
    yc = y - mu[None, :, None]
    var = jnp.sum(jnp.sum(yc * yc, axis=2), axis=0) * (1.0 / cnt)
    scale = g_ref[...][0, :, 0] * jax.lax.rsqrt(var + 1e-5)
    out_ref[...] = yc * scale[None, :, None] + b_ref[...]


def _batch_norm(y, gamma, beta):
    n, c, f = y.shape
    return pl.pallas_call(
        _bn_body,
        out_shape=jax.ShapeDtypeStruct((n, c, f), jnp.float32),
    )(y, gamma.reshape(1, c, 1), beta.reshape(1, c, 1))


def _rgcn_layer(x, lin_w, att, rel_w, root_w, bias, relu_in, src, dst, et, fv):
    xl, init = _project(x, lin_w, root_w, bias, relu_in)
    logit = _edge_logits(xl, att, src, dst)
    aw = _segment_softmax(logit)
    return _messages(xl, aw, rel_w, init, src, et, fv, dst)


def kernel(vision_feat, vision_pos, motion_feat, motion_pos,
           text_word_features, text_pos, lin_w1, att1, rel_w1, root_w1,
           bias1, lin_w2, att2, rel_w2, root_w2, bias2, bn_gamma, bn_beta):
    bs, frames, c, h, w = vision_feat.shape
    target = h * w
    L = text_word_features.shape[-1]
    rep = (target + L - 1) // L
    txt = jnp.tile(text_word_features, (1, 1, rep))[:, :, :target]
    xraw = jnp.concatenate(
        [vision_feat.reshape(bs, frames, c, target),
         motion_feat.reshape(bs, frames, c, target),
         txt.reshape(bs, 1, c, target)], axis=1)

    xf = _prep(xraw).reshape(_N_NODES, c, target)

    src = jnp.asarray(_SRC)
    dst = jnp.asarray(_DST)
    et = jnp.asarray(_ET)
    fv = jnp.asarray(_FV)

    h1 = _rgcn_layer(xf, lin_w1, att1, rel_w1, root_w1, bias1, False,
                     src, dst, et, fv)
    h2 = _rgcn_layer(h1, lin_w2, att2, rel_w2, root_w2, bias2, True,
                     src, dst, et, fv)
    y = _batch_norm(h2, bn_gamma, bn_beta)
    return y.reshape(-1, c, h, w)


# speedup vs baseline: 8.2348x; 2.0776x over previous
import functools

import numpy as np
import jax
import jax.numpy as jnp
from jax.experimental import pallas as pl
from jax.experimental.pallas import tpu as pltpu

_H = 4
_NUM_REL = 7
_FRAMES = 4
_BS = 2
_M = 2 * _FRAMES + 1
_N_NODES = _BS * (_M + 1)


def _build_static_edges():
    ei = []
    et = []
    for b in range(_BS):
        start = b * _M
        total = start + _M
        first_flow = start + _FRAMES
        for img in range(start, first_flow):
            j = img - start
            if img < first_flow - 1:
                ei.append([img, img + 1]); ei.append([first_flow + j, first_flow + j + 1])
                et += [0, 1]
            ei.append([img, first_flow + j]); ei.append([first_flow + j, img]); et += [2, 2]
            ei.append([img, total - 1]); ei.append([total - 1, img]); et += [3, 3]
            ei.append([first_flow + j, total - 1]); ei.append([total - 1, first_flow + j]); et += [4, 4]
        g = total
        for node in range(start, total):
            ei.append([g, node]); ei.append([node, g]); et += [5, 5]
    ei = np.array(ei, dtype=np.int32).T
    n_used = int(ei.max()) + 1
    loops = np.tile(np.arange(n_used, dtype=np.int32), (2, 1))
    ei = np.concatenate([ei, loops], axis=1)
    et = np.concatenate([np.array(et, dtype=np.int32),
                         np.full((n_used,), 6, dtype=np.int32)])
    src, dst = ei[0], ei[1]

    cnt = np.zeros((_N_NODES, _NUM_REL), np.int64)
    for s, d, t in zip(src, dst, et):
        cnt[d, t] += 1
    norm = 1.0 / np.maximum(cnt[dst, et], 1)

    present = set(dst.tolist())
    extra = [d for d in range(_N_NODES) if d not in present]
    src = np.concatenate([src, np.array(extra, np.int32)])
    dst = np.concatenate([dst, np.array(extra, np.int32)])
    et = np.concatenate([et, np.zeros(len(extra), np.int32)])
    norm = np.concatenate([norm, np.zeros(len(extra))])

    order = np.argsort(dst, kind="stable")
    return (src[order].astype(np.int32), dst[order].astype(np.int32),
            et[order].astype(np.int32),
            (norm[order] / _H).astype(np.float32))


_SRC, _DST, _ET, _NORMH = _build_static_edges()
_E = len(_SRC)


def _layer_body(src_ref, dst_ref, et_ref, nrm_ref, x_ref, lin_ref, att_ref,
                relw_ref, root_ref, bias_ref, bng_ref, bnb_ref, out_ref,
                xl_ref, logit_ref, amax_ref, den_ref, *, o, f, mode):
    lin = lin_ref[...]
    root_t = root_ref[...].T
    bias = bias_ref[...]

    def project(n_id, xn):
        xl = jnp.dot(lin, xn, preferred_element_type=jnp.float32)
        xl = xl.reshape(_H, o, f)
        xl_ref[n_id] = xl
        xr = (xl[0] + xl[1] + xl[2] + xl[3]) * (1.0 / _H)
        out_ref[n_id] = jnp.dot(root_t, xr,
                                preferred_element_type=jnp.float32) + bias

    if mode == "first":
        for b in range(_BS):
            drows = [x_ref[b, i] + x_ref[b, i] for i in range(_M)]
            gsum = drows[0]
            for r in drows[1:]:
                gsum = gsum + r
            rows = drows + [gsum * (1.0 / _M)]
            for i, xn in enumerate(rows):
                project(b * (_M + 1) + i, xn)
    else:
        for n_id in range(_N_NODES):
            project(n_id, jnp.maximum(x_ref[n_id], 0.0))

    att = att_ref[...]

    def logits_step(e, _):
        s = src_ref[e]
        d = dst_ref[e]
        xs = xl_ref[pl.ds(s, 1)][0] + xl_ref[pl.ds(d, 1)][0]
        lr = jnp.where(xs >= 0, xs, 0.2 * xs)
        logit_ref[pl.ds(e, 1)] = jnp.sum(lr * att[:, :, None], axis=1)[None]
        return 0

    jax.lax.fori_loop(0, _E, logits_step, 0)

    amax_ref[...] = jnp.full((_N_NODES, _H, f), -1e30, jnp.float32)
    den_ref[...] = jnp.zeros((_N_NODES, _H, f), jnp.float32)

    def max_step(e, _):
        d = dst_ref[e]
        cur = amax_ref[pl.ds(d, 1)]
        amax_ref[pl.ds(d, 1)] = jnp.maximum(cur, logit_ref[pl.ds(e, 1)])
        return 0

    jax.lax.fori_loop(0, _E, max_step, 0)

    def exp_step(e, _):
        d = dst_ref[e]
        ex = jnp.exp(logit_ref[pl.ds(e, 1)] - amax_ref[pl.ds(d, 1)])
        logit_ref[pl.ds(e, 1)] = ex
        den_ref[pl.ds(d, 1)] = den_ref[pl.ds(d, 1)] + ex
        return 0

    jax.lax.fori_loop(0, _E, exp_step, 0)

    den_ref[...] = 1.0 / (den_ref[...] + 1e-16)

    def msg_step(e, _):
        s = src_ref[e]
        d = dst_ref[e]
        t = et_ref[e]
        aw = logit_ref[pl.ds(e, 1)][0] * den_ref[pl.ds(d, 1)][0] * nrm_ref[e]
        xj = xl_ref[pl.ds(s, 1)][0]
        m = xj[0] * aw[0][None, :]
        for hh in range(1, _H):
            m = m + xj[hh] * aw[hh][None, :]
        w = relw_ref[pl.ds(t, 1)][0]
        msg = jax.lax.dot_general(w, m, (((0,), (0,)), ((), ())),
                                  preferred_element_type=jnp.float32)
        out_ref[pl.ds(d, 1)] = out_ref[pl.ds(d, 1)] + msg[None]
        return 0

    jax.lax.fori_loop(0, _E, msg_step, 0)

    if mode == "second":
        y = out_ref[...]
        cnt = _N_NODES * f
        mu = jnp.sum(jnp.sum(y, axis=2), axis=0) * (1.0 / cnt)
        yc = y - mu[None, :, None]
        var = jnp.sum(jnp.sum(yc * yc, axis=2), axis=0) * (1.0 / cnt)
        scale = bng_ref[...][:, 0] * jax.lax.rsqrt(var + 1e-5)
        out_ref[...] = yc * scale[None, :, None] + bnb_ref[...][None]


def _run_layer(x, lin_w, att, rel_w, root_w, bias, bn_g, bn_b, mode):
    k, c = lin_w.shape
    o = k // _H
    f = x.shape[-1]
    body = functools.partial(_layer_body, o=o, f=f, mode=mode)
    smem = pl.BlockSpec(memory_space=pltpu.SMEM)
    vmem = pl.BlockSpec(memory_space=pltpu.VMEM)
    return pl.pallas_call(
        body,
        in_specs=[smem, smem, smem, smem, vmem, vmem, vmem, vmem, vmem, vmem,
                  vmem, vmem],
        out_specs=vmem,
        out_shape=jax.ShapeDtypeStruct((_N_NODES, o, f), jnp.float32),
        scratch_shapes=[
            pltpu.VMEM((_N_NODES, _H, o, f), jnp.float32),
            pltpu.VMEM((_E, _H, f), jnp.float32),
            pltpu.VMEM((_N_NODES, _H, f), jnp.float32),
            pltpu.VMEM((_N_NODES, _H, f), jnp.float32),
        ],
    )(jnp.asarray(_SRC), jnp.asarray(_DST), jnp.asarray(_ET),
      jnp.asarray(_NORMH), x, lin_w, att, rel_w, root_w, bias.reshape(o, 1),
      bn_g.reshape(o, 1), bn_b.reshape(o, 1))


def kernel(vision_feat, vision_pos, motion_feat, motion_pos,
           text_word_features, text_pos, lin_w1, att1, rel_w1, root_w1,
           bias1, lin_w2, att2, rel_w2, root_w2, bias2, bn_gamma, bn_beta):
    bs, frames, c, h, w = vision_feat.shape
    target = h * w
    L = text_word_features.shape[-1]
    rep = (target + L - 1) // L
    txt = jnp.tile(text_word_features, (1, 1, rep))[:, :, :target]
    xraw = jnp.concatenate(
        [vision_feat.reshape(bs, frames, c, target),
         motion_feat.reshape(bs, frames, c, target),
         txt.reshape(bs, 1, c, target)], axis=1)

    dummy = jnp.zeros((2 * c,), jnp.float32)
    h1 = _run_layer(xraw, lin_w1, att1, rel_w1, root_w1, bias1,
                    dummy, dummy, "first")
    y = _run_layer(h1, lin_w2, att2, rel_w2, root_w2, bias2,
                   bn_gamma, bn_beta, "second")
    return y.reshape(-1, c, h, w)


# speedup vs baseline: 9.1236x; 1.1079x over previous
import functools

import numpy as np
import jax
import jax.numpy as jnp
from jax.experimental import pallas as pl
from jax.experimental.pallas import tpu as pltpu

_H = 4
_NUM_REL = 7
_FRAMES = 4
_BS = 2
_M = 2 * _FRAMES + 1
_N_NODES = _BS * (_M + 1)


def _build_static_edges():
    ei = []
    et = []
    for b in range(_BS):
        start = b * _M
        total = start + _M
        first_flow = start + _FRAMES
        for img in range(start, first_flow):
            j = img - start
            if img < first_flow - 1:
                ei.append([img, img + 1]); ei.append([first_flow + j, first_flow + j + 1])
                et += [0, 1]
            ei.append([img, first_flow + j]); ei.append([first_flow + j, img]); et += [2, 2]
            ei.append([img, total - 1]); ei.append([total - 1, img]); et += [3, 3]
            ei.append([first_flow + j, total - 1]); ei.append([total - 1, first_flow + j]); et += [4, 4]
        g = total
        for node in range(start, total):
            ei.append([g, node]); ei.append([node, g]); et += [5, 5]
    ei = np.array(ei, dtype=np.int32).T
    n_used = int(ei.max()) + 1
    loops = np.tile(np.arange(n_used, dtype=np.int32), (2, 1))
    ei = np.concatenate([ei, loops], axis=1)
    et = np.concatenate([np.array(et, dtype=np.int32),
                         np.full((n_used,), 6, dtype=np.int32)])
    src, dst = ei[0], ei[1]

    cnt = np.zeros((_N_NODES, _NUM_REL), np.int64)
    for s, d, t in zip(src, dst, et):
        cnt[d, t] += 1
    norm = 1.0 / np.maximum(cnt[dst, et], 1)

    present = set(dst.tolist())
    extra = [d for d in range(_N_NODES) if d not in present]
    src = np.concatenate([src, np.array(extra, np.int32)])
    dst = np.concatenate([dst, np.array(extra, np.int32)])
    et = np.concatenate([et, np.zeros(len(extra), np.int32)])
    norm = np.concatenate([norm, np.zeros(len(extra))])

    order = np.argsort(dst, kind="stable")
    src, dst, et = src[order].astype(np.int32), dst[order].astype(np.int32), \
        et[order].astype(np.int32)
    normh = (norm[order] / _H).astype(np.float32)

    pairs = {}
    for e, (s, d) in enumerate(zip(src.tolist(), dst.tolist())):
        pairs.setdefault((min(s, d), max(s, d)), []).append(e)
    pa, pb, pe1, pe2 = [], [], [], []
    for (a, b), es in pairs.items():
        pa.append(a); pb.append(b)
        pe1.append(es[0]); pe2.append(es[-1])

    seg = []
    bounds = np.flatnonzero(np.concatenate([[1], (dst[1:] != dst[:-1]), [1]]))
    for i in range(len(bounds) - 1):
        seg.append((int(bounds[i]), int(bounds[i + 1] - bounds[i])))

    return (src, dst, et, normh,
            np.array(pa, np.int32), np.array(pb, np.int32),
            np.array(pe1, np.int32), np.array(pe2, np.int32), seg)


(_SRC, _DST, _ET, _NORMH, _PA, _PB, _PE1, _PE2, _SEG) = _build_static_edges()
_E = len(_SRC)
_P = len(_PA)


def _layer_body(src_ref, dst_ref, et_ref, pa_ref, pb_ref, pe1_ref, pe2_ref,
                x_ref, lin_ref, attbd_ref, relwb_ref, root_ref, bias_ref,
                bng_ref, bnb_ref, out_ref, xl_ref, xlb_ref, logit_ref,
                awb_ref, *, o, f, mode):
    lin = lin_ref[...]
    root_t = root_ref[...].T
    bias = bias_ref[...]

    def project(n_id, xn):
        xl = jnp.dot(lin, xn, preferred_element_type=jnp.float32)
        xl = xl.reshape(_H, o, f)
        xl_ref[n_id] = xl
        xlb_ref[n_id] = xl.astype(jnp.bfloat16)
        xr = (xl[0] + xl[1] + xl[2] + xl[3]) * (1.0 / _H)
        out_ref[n_id] = jnp.dot(root_t, xr,
                                preferred_element_type=jnp.float32) + bias

    if mode == "first":
        for b in range(_BS):
            drows = [x_ref[b, i] + x_ref[b, i] for i in range(_M)]
            gsum = drows[0]
            for r in drows[1:]:
                gsum = gsum + r
            rows = drows + [gsum * (1.0 / _M)]
            for i, xn in enumerate(rows):
                project(b * (_M + 1) + i, xn)
    else:
        for n_id in range(_N_NODES):
            project(n_id, jnp.maximum(x_ref[n_id], 0.0))

    attbd = attbd_ref[...]

    def logits_step(p, _):
        a = pa_ref[p]
        b = pb_ref[p]
        xs = xl_ref[pl.ds(a, 1)][0] + xl_ref[pl.ds(b, 1)][0]
        lr = jnp.where(xs >= 0, xs, 0.2 * xs).reshape(_H * o, f)
        lg = jnp.dot(attbd, lr, preferred_element_type=jnp.float32)[None]
        logit_ref[pl.ds(pe1_ref[p], 1)] = lg
        logit_ref[pl.ds(pe2_ref[p], 1)] = lg
        return 0

    jax.lax.fori_loop(0, _P, logits_step, 0)

    for s0, cnt in _SEG:
        rows = [logit_ref[s0 + i] for i in range(cnt)]
        amax = rows[0]
        for r in rows[1:]:
            amax = jnp.maximum(amax, r)
        exs = [jnp.exp(r - amax) for r in rows]
        den = exs[0]
        for r in exs[1:]:
            den = den + r
        inv = 1.0 / (den + 1e-16)
        for i in range(cnt):
            awb_ref[s0 + i] = (exs[i] * inv *
                               float(_NORMH[s0 + i])).astype(jnp.bfloat16)

    def msg_step(e, _):
        s = src_ref[e]
        d = dst_ref[e]
        t = et_ref[e]
        aw = awb_ref[pl.ds(e, 1)][0]
        xj = xlb_ref[pl.ds(s, 1)][0]
        m = xj[0] * aw[0][None, :]
        for hh in range(1, _H):
            m = m + xj[hh] * aw[hh][None, :]
        w = relwb_ref[pl.ds(t, 1)][0]
        msg = jax.lax.dot_general(w, m, (((0,), (0,)), ((), ())),
                                  preferred_element_type=jnp.float32)
        out_ref[pl.ds(d, 1)] = out_ref[pl.ds(d, 1)] + msg[None]
        return 0

    jax.lax.fori_loop(0, _E, msg_step, 0)

    if mode == "second":
        y = out_ref[...]
        cnt = _N_NODES * f
        mu = jnp.sum(jnp.sum(y, axis=2), axis=0) * (1.0 / cnt)
        yc = y - mu[None, :, None]
        var = jnp.sum(jnp.sum(yc * yc, axis=2), axis=0) * (1.0 / cnt)
        scale = bng_ref[...][:, 0] * jax.lax.rsqrt(var + 1e-5)
        out_ref[...] = yc * scale[None, :, None] + bnb_ref[...][None]


def _run_layer(x, lin_w, att, rel_w, root_w, bias, bn_g, bn_b, mode):
    k, c = lin_w.shape
    o = k // _H
    f = x.shape[-1]
    body = functools.partial(_layer_body, o=o, f=f, mode=mode)
    smem = pl.BlockSpec(memory_space=pltpu.SMEM)
    vmem = pl.BlockSpec(memory_space=pltpu.VMEM)
    attbd = jnp.zeros((_H, _H * o), jnp.float32)
    for hh in range(_H):
        attbd = attbd.at[hh, hh * o:(hh + 1) * o].set(att[hh])
    return pl.pallas_call(
        body,
        in_specs=[smem] * 7 + [vmem] * 8,
        out_specs=vmem,
        out_shape=jax.ShapeDtypeStruct((_N_NODES, o, f), jnp.float32),
        scratch_shapes=[
            pltpu.VMEM((_N_NODES, _H, o, f), jnp.float32),
            pltpu.VMEM((_N_NODES, _H, o, f), jnp.bfloat16),
            pltpu.VMEM((_E, _H, f), jnp.float32),
            pltpu.VMEM((_E, _H, f), jnp.bfloat16),
        ],
    )(jnp.asarray(_SRC), jnp.asarray(_DST), jnp.asarray(_ET),
      jnp.asarray(_PA), jnp.asarray(_PB), jnp.asarray(_PE1), jnp.asarray(_PE2),
      x, lin_w, attbd, rel_w.astype(jnp.bfloat16), root_w,
      bias.reshape(o, 1), bn_g.reshape(o, 1), bn_b.reshape(o, 1))


def kernel(vision_feat, vision_pos, motion_feat, motion_pos,
           text_word_features, text_pos, lin_w1, att1, rel_w1, root_w1,
           bias1, lin_w2, att2, rel_w2, root_w2, bias2, bn_gamma, bn_beta):
    bs, frames, c, h, w = vision_feat.shape
    target = h * w
    L = text_word_features.shape[-1]
    rep = (target + L - 1) // L
    txt = jnp.tile(text_word_features, (1, 1, rep))[:, :, :target]
    xraw = jnp.concatenate(
        [vision_feat.reshape(bs, frames, c, target),
         motion_feat.reshape(bs, frames, c, target),
         txt.reshape(bs, 1, c, target)], axis=1)

    dummy = jnp.zeros((2 * c,), jnp.float32)
    h1 = _run_layer(xraw, lin_w1, att1, rel_w1, root_w1, bias1,
                    dummy, dummy, "first")
    y = _run_layer(h1, lin_w2, att2, rel_w2, root_w2, bias2,
                   bn_gamma, bn_beta, "second")
    return y.reshape(-1, c, h, w)


# speedup vs baseline: 10.3794x; 1.1377x over previous
import functools

import numpy as np
import jax
import jax.numpy as jnp
from jax.experimental import pallas as pl
from jax.experimental.pallas import tpu as pltpu

_H = 4
_NUM_REL = 7
_FRAMES = 4
_BS = 2
_M = 2 * _FRAMES + 1
_N_NODES = _BS * (_M + 1)


def _build_static_edges():
    ei = []
    et = []
    for b in range(_BS):
        start = b * _M
        total = start + _M
        first_flow = start + _FRAMES
        for img in range(start, first_flow):
            j = img - start
            if img < first_flow - 1:
                ei.append([img, img + 1]); ei.append([first_flow + j, first_flow + j + 1])
                et += [0, 1]
            ei.append([img, first_flow + j]); ei.append([first_flow + j, img]); et += [2, 2]
            ei.append([img, total - 1]); ei.append([total - 1, img]); et += [3, 3]
            ei.append([first_flow + j, total - 1]); ei.append([total - 1, first_flow + j]); et += [4, 4]
        g = total
        for node in range(start, total):
            ei.append([g, node]); ei.append([node, g]); et += [5, 5]
    ei = np.array(ei, dtype=np.int32).T
    n_used = int(ei.max()) + 1
    loops = np.tile(np.arange(n_used, dtype=np.int32), (2, 1))
    ei = np.concatenate([ei, loops], axis=1)
    et = np.concatenate([np.array(et, dtype=np.int32),
                         np.full((n_used,), 6, dtype=np.int32)])
    src, dst = ei[0], ei[1]

    cnt = np.zeros((_N_NODES, _NUM_REL), np.int64)
    for s, d, t in zip(src, dst, et):
        cnt[d, t] += 1
    norm = 1.0 / np.maximum(cnt[dst, et], 1)

    present = set(dst.tolist())
    extra = [d for d in range(_N_NODES) if d not in present]
    src = np.concatenate([src, np.array(extra, np.int32)])
    dst = np.concatenate([dst, np.array(extra, np.int32)])
    et = np.concatenate([et, np.zeros(len(extra), np.int32)])
    norm = np.concatenate([norm, np.zeros(len(extra))])

    order = np.argsort(dst, kind="stable")
    src, dst, et = src[order].astype(np.int32), dst[order].astype(np.int32), \
        et[order].astype(np.int32)
    normh = (norm[order] / _H).astype(np.float32)

    pairs = {}
    for e, (s, d) in enumerate(zip(src.tolist(), dst.tolist())):
        pairs.setdefault((min(s, d), max(s, d)), []).append(e)
    pa, pb, pe1, pe2 = [], [], [], []
    for (a, b), es in pairs.items():
        pa.append(a); pb.append(b)
        pe1.append(es[0]); pe2.append(es[-1])

    seg = []
    bounds = np.flatnonzero(np.concatenate([[1], (dst[1:] != dst[:-1]), [1]]))
    for i in range(len(bounds) - 1):
        seg.append((int(bounds[i]), int(bounds[i + 1] - bounds[i])))

    return (src, dst, et, normh,
            np.array(pa, np.int32), np.array(pb, np.int32),
            np.array(pe1, np.int32), np.array(pe2, np.int32), seg)


(_SRC, _DST, _ET, _NORMH, _PA, _PB, _PE1, _PE2, _SEG) = _build_static_edges()
_E = len(_SRC)
_P = len(_PA)


def _layer_body(src_ref, dst_ref, et_ref, pa_ref, pb_ref, pe1_ref, pe2_ref,
                x_ref, lin_ref, linb_ref, attbd_ref, relwb_ref, root_ref,
                bias_ref, bng_ref, bnb_ref, out_ref, xlb_ref, logit_ref,
                awb_ref, *, o, f, mode):
    lin_h = lin_ref[...].reshape(_H, o, lin_ref.shape[1])
    mavg = (lin_h[0] + lin_h[1] + lin_h[2] + lin_h[3]) * (1.0 / _H)
    rmat = jnp.dot(root_ref[...].T, mavg,
                   preferred_element_type=jnp.float32)

    linb = linb_ref[...]
    bias = bias_ref[...]

    def project(n_id, xn):
        xl = jnp.dot(linb, xn.astype(jnp.bfloat16),
                     preferred_element_type=jnp.float32)
        xlb_ref[n_id] = xl.reshape(_H, o, f).astype(jnp.bfloat16)
        out_ref[n_id] = jnp.dot(rmat, xn,
                                preferred_element_type=jnp.float32) + bias

    if mode == "first":
        for b in range(_BS):
            drows = [x_ref[b, i] + x_ref[b, i] for i in range(_M)]
            gsum = drows[0]
            for r in drows[1:]:
                gsum = gsum + r
            rows = drows + [gsum * (1.0 / _M)]
            for i, xn in enumerate(rows):
                project(b * (_M + 1) + i, xn)
    else:
        for n_id in range(_N_NODES):
            project(n_id, jnp.maximum(x_ref[n_id], 0.0))

    attbd = attbd_ref[...]

    def logits_step(p, _):
        a = pa_ref[p]
        b = pb_ref[p]
        xs = xlb_ref[pl.ds(a, 1)][0] + xlb_ref[pl.ds(b, 1)][0]
        lr = jnp.where(xs >= 0, xs, jnp.bfloat16(0.2) * xs).reshape(_H * o, f)
        lg = jnp.dot(attbd, lr, preferred_element_type=jnp.float32)[None]
        logit_ref[pl.ds(pe1_ref[p], 1)] = lg
        logit_ref[pl.ds(pe2_ref[p], 1)] = lg
        return 0

    jax.lax.fori_loop(0, _P, logits_step, 0)

    for s0, cnt in _SEG:
        rows = [logit_ref[s0 + i] for i in range(cnt)]
        amax = rows[0]
        for r in rows[1:]:
            amax = jnp.maximum(amax, r)
        exs = [jnp.exp(r - amax) for r in rows]
        den = exs[0]
        for r in exs[1:]:
            den = den + r
        inv = 1.0 / (den + 1e-16)
        for i in range(cnt):
            awb_ref[s0 + i] = (exs[i] * inv *
                               float(_NORMH[s0 + i])).astype(jnp.bfloat16)

    def msg_step(e, _):
        s = src_ref[e]
        d = dst_ref[e]
        t = et_ref[e]
        aw = awb_ref[pl.ds(e, 1)][0]
        xj = xlb_ref[pl.ds(s, 1)][0]
        m = xj[0] * aw[0][None, :]
        for hh in range(1, _H):
            m = m + xj[hh] * aw[hh][None, :]
        w = relwb_ref[pl.ds(t, 1)][0]
        msg = jax.lax.dot_general(w, m, (((0,), (0,)), ((), ())),
                                  preferred_element_type=jnp.float32)
        out_ref[pl.ds(d, 1)] = out_ref[pl.ds(d, 1)] + msg[None]
        return 0

    jax.lax.fori_loop(0, _E, msg_step, 0)

    if mode == "second":
        y = out_ref[...]
        cnt = _N_NODES * f
        mu = jnp.sum(jnp.sum(y, axis=2), axis=0) * (1.0 / cnt)
        yc = y - mu[None, :, None]
        var = jnp.sum(jnp.sum(yc * yc, axis=2), axis=0) * (1.0 / cnt)
        scale = bng_ref[...][:, 0] * jax.lax.rsqrt(var + 1e-5)
        out_ref[...] = yc * scale[None, :, None] + bnb_ref[...][None]


def _run_layer(x, lin_w, att, rel_w, root_w, bias, bn_g, bn_b, mode):
    k, c = lin_w.shape
    o = k // _H
    f = x.shape[-1]
    body = functools.partial(_layer_body, o=o, f=f, mode=mode)
    smem = pl.BlockSpec(memory_space=pltpu.SMEM)
    vmem = pl.BlockSpec(memory_space=pltpu.VMEM)
    attbd = jnp.zeros((_H, _H * o), jnp.float32)
    for hh in range(_H):
        attbd = attbd.at[hh, hh * o:(hh + 1) * o].set(att[hh])
    return pl.pallas_call(
        body,
        in_specs=[smem] * 7 + [vmem] * 9,
        out_specs=vmem,
        out_shape=jax.ShapeDtypeStruct((_N_NODES, o, f), jnp.float32),
        scratch_shapes=[
            pltpu.VMEM((_N_NODES, _H, o, f), jnp.bfloat16),
            pltpu.VMEM((_E, _H, f), jnp.float32),
            pltpu.VMEM((_E, _H, f), jnp.bfloat16),
        ],
    )(jnp.asarray(_SRC), jnp.asarray(_DST), jnp.asarray(_ET),
      jnp.asarray(_PA), jnp.asarray(_PB), jnp.asarray(_PE1), jnp.asarray(_PE2),
      x, lin_w, lin_w.astype(jnp.bfloat16), attbd.astype(jnp.bfloat16),
      rel_w.astype(jnp.bfloat16), root_w,
      bias.reshape(o, 1), bn_g.reshape(o, 1), bn_b.reshape(o, 1))


def kernel(vision_feat, vision_pos, motion_feat, motion_pos,
           text_word_features, text_pos, lin_w1, att1, rel_w1, root_w1,
           bias1, lin_w2, att2, rel_w2, root_w2, bias2, bn_gamma, bn_beta):
    bs, frames, c, h, w = vision_feat.shape
    target = h * w
    L = text_word_features.shape[-1]
    rep = (target + L - 1) // L
    txt = jnp.tile(text_word_features, (1, 1, rep))[:, :, :target]
    xraw = jnp.concatenate(
        [vision_feat.reshape(bs, frames, c, target),
         motion_feat.reshape(bs, frames, c, target),
         txt.reshape(bs, 1, c, target)], axis=1)

    dummy = jnp.zeros((2 * c,), jnp.float32)
    h1 = _run_layer(xraw, lin_w1, att1, rel_w1, root_w1, bias1,
                    dummy, dummy, "first")
    y = _run_layer(h1, lin_w2, att2, rel_w2, root_w2, bias2,
                   bn_gamma, bn_beta, "second")
    return y.reshape(-1, c, h, w)


# speedup vs baseline: 16.8995x; 1.6282x over previous
import functools

import numpy as np
import jax
import jax.numpy as jnp
from jax.experimental import pallas as pl
from jax.experimental.pallas import tpu as pltpu

_H = 4
_NUM_REL = 7
_FRAMES = 4
_BS = 2
_M = 2 * _FRAMES + 1
_N_NODES = _BS * (_M + 1)


def _build_static_edges():
    ei = []
    et = []
    for b in range(_BS):
        start = b * _M
        total = start + _M
        first_flow = start + _FRAMES
        for img in range(start, first_flow):
            j = img - start
            if img < first_flow - 1:
                ei.append([img, img + 1]); ei.append([first_flow + j, first_flow + j + 1])
                et += [0, 1]
            ei.append([img, first_flow + j]); ei.append([first_flow + j, img]); et += [2, 2]
            ei.append([img, total - 1]); ei.append([total - 1, img]); et += [3, 3]
            ei.append([first_flow + j, total - 1]); ei.append([total - 1, first_flow + j]); et += [4, 4]
        g = total
        for node in range(start, total):
            ei.append([g, node]); ei.append([node, g]); et += [5, 5]
    ei = np.array(ei, dtype=np.int32).T
    n_used = int(ei.max()) + 1
    loops = np.tile(np.arange(n_used, dtype=np.int32), (2, 1))
    ei = np.concatenate([ei, loops], axis=1)
    et = np.concatenate([np.array(et, dtype=np.int32),
                         np.full((n_used,), 6, dtype=np.int32)])
    src, dst = ei[0], ei[1]

    cnt = np.zeros((_N_NODES, _NUM_REL), np.int64)
    for s, d, t in zip(src, dst, et):
        cnt[d, t] += 1
    norm = 1.0 / np.maximum(cnt[dst, et], 1)

    present = set(dst.tolist())
    extra = [d for d in range(_N_NODES) if d not in present]
    src = np.concatenate([src, np.array(extra, np.int32)])
    dst = np.concatenate([dst, np.array(extra, np.int32)])
    et = np.concatenate([et, np.zeros(len(extra), np.int32)])
    norm = np.concatenate([norm, np.zeros(len(extra))])

    order = np.argsort(dst, kind="stable")
    src, dst, et = src[order].astype(np.int32), dst[order].astype(np.int32), \
        et[order].astype(np.int32)
    normh = (norm[order] / _H).astype(np.float32)

    pairs = {}
    for e, (s, d) in enumerate(zip(src.tolist(), dst.tolist())):
        pairs.setdefault((min(s, d), max(s, d)), []).append(e)
    pa, pb, pe1, pe2 = [], [], [], []
    for (a, b), es in pairs.items():
        pa.append(a); pb.append(b)
        pe1.append(es[0]); pe2.append(es[-1])
    while len(pa) % 4:
        pa.append(pa[-1]); pb.append(pb[-1])
        pe1.append(pe1[-1]); pe2.append(pe2[-1])

    seg = []
    bounds = np.flatnonzero(np.concatenate([[1], (dst[1:] != dst[:-1]), [1]]))
    for i in range(len(bounds) - 1):
        seg.append((int(bounds[i]), int(bounds[i + 1] - bounds[i])))

    return (src, dst, et, normh,
            np.array(pa, np.int32), np.array(pb, np.int32),
            np.array(pe1, np.int32), np.array(pe2, np.int32), seg)


(_SRC, _DST, _ET, _NORMH, _PA, _PB, _PE1, _PE2, _SEG) = _build_static_edges()
_E = len(_SRC)
_P = len(_PA)


def _layer_body(src_ref, dst_ref, et_ref, pa_ref, pb_ref, pe1_ref, pe2_ref,
                x_ref, lin_ref, linb_ref, attbd_ref, relwb_ref, root_ref,
                bias_ref, bng_ref, bnb_ref, out_ref, xlb_ref, logit_ref,
                awb_ref, *, o, f, mode):
    lin_h = lin_ref[...].reshape(_H, o, lin_ref.shape[1])
    mavg = (lin_h[0] + lin_h[1] + lin_h[2] + lin_h[3]) * (1.0 / _H)
    rmat = jnp.dot(root_ref[...].T, mavg,
                   preferred_element_type=jnp.float32)

    linb = linb_ref[...]
    bias = bias_ref[...]

    def project(n_id, xn):
        xl = jnp.dot(linb, xn.astype(jnp.bfloat16),
                     preferred_element_type=jnp.float32)
        xlb_ref[n_id] = xl.reshape(_H, o, f).astype(jnp.bfloat16)
        out_ref[n_id] = jnp.dot(rmat, xn,
                                preferred_element_type=jnp.float32) + bias

    if mode == "first":
        for b in range(_BS):
            drows = [x_ref[b, i] + x_ref[b, i] for i in range(_M)]
            gsum = drows[0]
            for r in drows[1:]:
                gsum = gsum + r
            rows = drows + [gsum * (1.0 / _M)]
            for i, xn in enumerate(rows):
                project(b * (_M + 1) + i, xn)
    else:
        for n_id in range(_N_NODES):
            project(n_id, jnp.maximum(x_ref[n_id], 0.0))

    attbd = attbd_ref[...]

    def logits_quad(q, _):
        for k in range(4):
            p = q * 4 + k
            a = pa_ref[p]
            b = pb_ref[p]
            xs = xlb_ref[pl.ds(a, 1)][0] + xlb_ref[pl.ds(b, 1)][0]
            lr = jnp.where(xs >= 0, xs, jnp.bfloat16(0.2) * xs).reshape(_H * o, f)
            lg = jnp.dot(attbd, lr, preferred_element_type=jnp.float32)[None]
            logit_ref[pl.ds(pe1_ref[p], 1)] = lg
            logit_ref[pl.ds(pe2_ref[p], 1)] = lg
        return 0

    jax.lax.fori_loop(0, _P // 4, logits_quad, 0)

    for s0, cnt in _SEG:
        rows = [logit_ref[s0 + i] for i in range(cnt)]
        amax = rows[0]
        for r in rows[1:]:
            amax = jnp.maximum(amax, r)
        exs = [jnp.exp(r - amax) for r in rows]
        den = exs[0]
        for r in exs[1:]:
            den = den + r
        inv = 1.0 / (den + 1e-16)
        for i in range(cnt):
            awb_ref[s0 + i] = (exs[i] * inv *
                               float(_NORMH[s0 + i])).astype(jnp.bfloat16)

    def msg_quad(q, _):
        for k in range(4):
            e = q * 4 + k
            s = src_ref[e]
            d = dst_ref[e]
            t = et_ref[e]
            aw = awb_ref[pl.ds(e, 1)][0]
            xj = xlb_ref[pl.ds(s, 1)][0]
            m = xj[0] * aw[0][None, :]
            for hh in range(1, _H):
                m = m + xj[hh] * aw[hh][None, :]
            w = relwb_ref[pl.ds(t, 1)][0]
            msg = jax.lax.dot_general(w, m, (((0,), (0,)), ((), ())),
                                      preferred_element_type=jnp.float32)
            out_ref[pl.ds(d, 1)] = out_ref[pl.ds(d, 1)] + msg[None]
        return 0

    jax.lax.fori_loop(0, _E // 4, msg_quad, 0)

    if mode == "second":
        y = out_ref[...]
        cnt = _N_NODES * f
        mu = jnp.sum(jnp.sum(y, axis=2), axis=0) * (1.0 / cnt)
        yc = y - mu[None, :, None]
        var = jnp.sum(jnp.sum(yc * yc, axis=2), axis=0) * (1.0 / cnt)
        scale = bng_ref[...][:, 0] * jax.lax.rsqrt(var + 1e-5)
        out_ref[...] = yc * scale[None, :, None] + bnb_ref[...][None]


def _run_layer(x, lin_w, att, rel_w, root_w, bias, bn_g, bn_b, mode):
    k, c = lin_w.shape
    o = k // _H
    f = x.shape[-1]
    body = functools.partial(_layer_body, o=o, f=f, mode=mode)
    smem = pl.BlockSpec(memory_space=pltpu.SMEM)
    vmem = pl.BlockSpec(memory_space=pltpu.VMEM)
    attbd = jnp.zeros((_H, _H * o), jnp.float32)
    for hh in range(_H):
        attbd = attbd.at[hh, hh * o:(hh + 1) * o].set(att[hh])
    return pl.pallas_call(
        body,
        in_specs=[smem] * 7 + [vmem] * 9,
        out_specs=vmem,
        out_shape=jax.ShapeDtypeStruct((_N_NODES, o, f), jnp.float32),
        scratch_shapes=[
            pltpu.VMEM((_N_NODES, _H, o, f), jnp.bfloat16),
            pltpu.VMEM((_E, _H, f), jnp.float32),
            pltpu.VMEM((_E, _H, f), jnp.bfloat16),
        ],
    )(jnp.asarray(_SRC), jnp.asarray(_DST), jnp.asarray(_ET),
      jnp.asarray(_PA), jnp.asarray(_PB), jnp.asarray(_PE1), jnp.asarray(_PE2),
      x, lin_w, lin_w.astype(jnp.bfloat16), attbd.astype(jnp.bfloat16),
      rel_w.astype(jnp.bfloat16), root_w,
      bias.reshape(o, 1), bn_g.reshape(o, 1), bn_b.reshape(o, 1))


def kernel(vision_feat, vision_pos, motion_feat, motion_pos,
           text_word_features, text_pos, lin_w1, att1, rel_w1, root_w1,
           bias1, lin_w2, att2, rel_w2, root_w2, bias2, bn_gamma, bn_beta):
    bs, frames, c, h, w = vision_feat.shape
    target = h * w
    L = text_word_features.shape[-1]
    rep = (target + L - 1) // L
    txt = jnp.tile(text_word_features, (1, 1, rep))[:, :, :target]
    xraw = jnp.concatenate(
        [vision_feat.reshape(bs, frames, c, target),
         motion_feat.reshape(bs, frames, c, target),
         txt.reshape(bs, 1, c, target)], axis=1)

    dummy = jnp.zeros((2 * c,), jnp.float32)
    h1 = _run_layer(xraw, lin_w1, att1, rel_w1, root_w1, bias1,
                    dummy, dummy, "first")
    y = _run_layer(h1, lin_w2, att2, rel_w2, root_w2, bias2,
                   bn_gamma, bn_beta, "second")
    return y.reshape(-1, c, h, w)
